# Initial kernel scaffold; baseline (speedup 1.0000x reference)
import jax
import jax.numpy as jnp
from jax.experimental import pallas as pl


def kernel(z, codebook, mask):
    raise NotImplementedError("write your pallas kernel here")


# speedup vs baseline: 1.0287x; 1.0287x over previous
import jax
import jax.numpy as jnp
from jax import lax
from jax.experimental import pallas as pl
from jax.experimental.pallas import tpu as pltpu
from jax.experimental.pallas import tpu_sc as plsc

NUM_CODES = 8192
DIM = 256
N_TOK = 16384
TOK_BLK = 512
CODE_BLK = 1024

_NC = 2
_NS = 16
_NW = _NC * _NS
_ROWS_PER_W = N_TOK // _NW
_CHUNK = 128
_NCHUNK = _ROWS_PER_W // _CHUNK


WIN = 2736
N_WIN = 3


def _argmin_body(z_ref, e_ref, e2_ref, z2_ref, idx_out, dsum_out):
    t = pl.program_id(0)
    z = z_ref[...]
    z2 = z2_ref[...]
    BIG = jnp.int32(2**30)
    wmin = [jnp.full((TOK_BLK, 1), jnp.inf, jnp.float32) for _ in range(N_WIN)]
    widx = [jnp.full((TOK_BLK, 1), BIG) for _ in range(N_WIN)]
    for c in range(NUM_CODES // CODE_BLK):
        lo, hi = c * CODE_BLK, (c + 1) * CODE_BLK
        e_blk = e_ref[lo:hi, :]
        e2_blk = e2_ref[:, lo:hi]
        m = lax.dot_general(z, e_blk, (((1,), (1,)), ((), ())),
                            preferred_element_type=jnp.float32)
        d = (z2 + e2_blk) - 2.0 * m
        cols = lax.broadcasted_iota(jnp.int32, (TOK_BLK, CODE_BLK), 1) + lo
        for w in range(N_WIN):
            w_lo, w_hi = w * WIN, min((w + 1) * WIN, NUM_CODES)
            if w_hi <= lo or w_lo >= hi:
                continue
            if w_lo <= lo and hi <= w_hi:
                dm = d
            else:
                in_w = (cols >= w_lo) & (cols < w_hi)
                dm = jnp.where(in_w, d, jnp.inf)
            loc_min = jnp.min(dm, axis=1, keepdims=True)
            loc_idx = jnp.min(jnp.where(dm == loc_min, cols, BIG),
                              axis=1, keepdims=True)
            better = loc_min < wmin[w]
            widx[w] = jnp.where(better, loc_idx, widx[w])
            wmin[w] = jnp.where(better, loc_min, wmin[w])
    acc_v, acc_i = wmin[0], widx[0]
    for w in range(1, N_WIN):
        accq = acc_v.astype(jnp.bfloat16).astype(jnp.float32)
        keep = (accq < wmin[w]) | ((accq == wmin[w]) & (acc_i < widx[w]))
        acc_i = jnp.where(keep, acc_i, widx[w])
        acc_v = jnp.where(keep, accq, wmin[w])
    run_min, run_idx = acc_v, acc_i
    idx_out[...] = run_idx

    @pl.when(t == 0)
    def _():
        dsum_out[...] = jnp.zeros((1, 1), jnp.float32)

    dsum_out[...] += jnp.sum(run_min, keepdims=True)


def _sc_gather_body(table_hbm, idx_hbm, out_hbm, idx_v, rows_v, sem):
    cid = lax.axis_index("c")
    sid = lax.axis_index("s")
    wid = sid * _NC + cid
    for j in range(_NCHUNK):
        base = wid * _ROWS_PER_W + j * _CHUNK
        pltpu.sync_copy(idx_hbm.at[pl.ds(base, _CHUNK)], idx_v)
        pltpu.async_copy(table_hbm.at[idx_v], rows_v, sem).wait()
        pltpu.sync_copy(rows_v, out_hbm.at[pl.ds(base, _CHUNK)])


def kernel(z, codebook, mask):
    e = lax.stop_gradient(codebook)
    z_flat = z.reshape(-1, DIM)
    e2 = jnp.sum(e ** 2, axis=1).reshape(1, NUM_CODES)
    z2 = jnp.sum(z_flat ** 2, axis=1, keepdims=True)

    nt = N_TOK // TOK_BLK
    min_idx_col, dsum = pl.pallas_call(
        _argmin_body,
        grid=(nt,),
        in_specs=[
            pl.BlockSpec((TOK_BLK, DIM), lambda t: (t, 0)),
            pl.BlockSpec((NUM_CODES, DIM), lambda t: (0, 0)),
            pl.BlockSpec((1, NUM_CODES), lambda t: (0, 0)),
            pl.BlockSpec((TOK_BLK, 1), lambda t: (t, 0)),
        ],
        out_specs=[
            pl.BlockSpec((TOK_BLK, 1), lambda t: (t, 0)),
            pl.BlockSpec((1, 1), lambda t: (0, 0)),
        ],
        out_shape=[
            jax.ShapeDtypeStruct((N_TOK, 1), jnp.int32),
            jax.ShapeDtypeStruct((1, 1), jnp.float32),
        ],
    )(z_flat, e, e2, z2)

    idx_flat = min_idx_col.reshape(N_TOK)

    sc_gather = pl.kernel(
        _sc_gather_body,
        out_type=jax.ShapeDtypeStruct((N_TOK, DIM), jnp.float32),
        mesh=plsc.VectorSubcoreMesh(core_axis_name="c", subcore_axis_name="s"),
        scratch_types=[
            pltpu.VMEM((_CHUNK,), jnp.int32),
            pltpu.VMEM((_CHUNK, DIM), jnp.float32),
            pltpu.SemaphoreType.DMA,
        ],
    )
    z_q = sc_gather(e, idx_flat).reshape(z.shape)

    valid_count = jnp.sum(mask) * z.shape[-1]
    commitment_loss = dsum[0, 0] / valid_count
    min_idx = idx_flat.reshape(z.shape[0], z.shape[1])
    return commitment_loss, z_q, min_idx


# speedup vs baseline: 1.3421x; 1.3047x over previous
import jax
import jax.numpy as jnp
from jax import lax
from jax.experimental import pallas as pl
from jax.experimental.pallas import tpu as pltpu
from jax.experimental.pallas import tpu_sc as plsc

NUM_CODES = 8192
DIM = 256
N_TOK = 16384
TOK_BLK = 512
CODE_BLK = 1024

_NC = 2
_NS = 16
_NW = _NC * _NS
_ROWS_PER_W = N_TOK // _NW
_CHUNK = 128
_NCHUNK = _ROWS_PER_W // _CHUNK


WIN = 2736
N_WIN = 3


def _argmin_body(z_ref, es_ref, e2_ref, z2_ref, cf_ref, idx_out, dsum_out, d_buf):
    t = pl.program_id(0)
    z = z_ref[...]
    z2 = z2_ref[...]
    INF = jnp.float32(jnp.inf)

    wmin = [jnp.full((TOK_BLK, 1), INF) for _ in range(N_WIN)]
    for c in range(NUM_CODES // CODE_BLK):
        lo, hi = c * CODE_BLK, (c + 1) * CODE_BLK
        m2 = lax.dot_general(z, es_ref[lo:hi, :], (((1,), (1,)), ((), ())),
                             preferred_element_type=jnp.float32)
        d = (z2 + e2_ref[:, lo:hi]) + m2
        d_buf[:, lo:hi] = d
        w_first, w_last = lo // WIN, (hi - 1) // WIN
        if w_first == w_last:
            loc = jnp.min(d, axis=1, keepdims=True)
            wmin[w_first] = jnp.minimum(wmin[w_first], loc)
        else:
            split = w_last * WIN
            cols = lax.broadcasted_iota(jnp.int32, (TOK_BLK, CODE_BLK), 1) + lo
            in_lo = cols < split
            loc_a = jnp.min(jnp.where(in_lo, d, INF), axis=1, keepdims=True)
            loc_b = jnp.min(jnp.where(in_lo, INF, d), axis=1, keepdims=True)
            wmin[w_first] = jnp.minimum(wmin[w_first], loc_a)
            wmin[w_last] = jnp.minimum(wmin[w_last], loc_b)

    win = jnp.zeros((TOK_BLK, 1), jnp.int32)
    acc = wmin[0]
    for w in range(1, N_WIN):
        accq = acc.astype(jnp.bfloat16).astype(jnp.float32)
        take = wmin[w] < accq
        win = jnp.where(take, jnp.int32(w), win)
        acc = jnp.where(take, wmin[w], accq)
    run_min = acc

    target = [jnp.where(win == w, wmin[w], INF) for w in range(N_WIN)]

    run_idx_f = jnp.full((TOK_BLK, 1), INF)
    for c in range(NUM_CODES // CODE_BLK):
        lo, hi = c * CODE_BLK, (c + 1) * CODE_BLK
        d = d_buf[:, lo:hi]
        colsf = cf_ref[:, lo:hi]
        w_first, w_last = lo // WIN, (hi - 1) // WIN
        if w_first == w_last:
            tgt = target[w_first]
        else:
            split = jnp.float32(w_last * WIN)
            tgt = jnp.where(colsf < split, target[w_first], target[w_last])
        cand = jnp.where(d == tgt, colsf, INF)
        run_idx_f = jnp.minimum(run_idx_f, jnp.min(cand, axis=1, keepdims=True))
    idx_out[...] = run_idx_f.astype(jnp.int32)

    @pl.when(t == 0)
    def _():
        dsum_out[...] = jnp.zeros((1, 1), jnp.float32)

    dsum_out[...] += jnp.sum(run_min, keepdims=True)


def _sc_gather_body(table_hbm, idx_hbm, out_hbm, idx_v, rows_v, sem):
    cid = lax.axis_index("c")
    sid = lax.axis_index("s")
    wid = sid * _NC + cid
    for j in range(_NCHUNK):
        base = wid * _ROWS_PER_W + j * _CHUNK
        pltpu.sync_copy(idx_hbm.at[pl.ds(base, _CHUNK)], idx_v)
        pltpu.async_copy(table_hbm.at[idx_v], rows_v, sem).wait()
        pltpu.sync_copy(rows_v, out_hbm.at[pl.ds(base, _CHUNK)])


def kernel(z, codebook, mask):
    e = lax.stop_gradient(codebook)
    z_flat = z.reshape(-1, DIM)
    es = e * jnp.float32(-2.0)
    e2 = jnp.sum(e ** 2, axis=1).reshape(1, NUM_CODES)
    z2 = jnp.sum(z_flat ** 2, axis=1, keepdims=True)
    colsf = lax.iota(jnp.float32, NUM_CODES).reshape(1, NUM_CODES)

    nt = N_TOK // TOK_BLK
    min_idx_col, dsum = pl.pallas_call(
        _argmin_body,
        grid=(nt,),
        in_specs=[
            pl.BlockSpec((TOK_BLK, DIM), lambda t: (t, 0)),
            pl.BlockSpec((NUM_CODES, DIM), lambda t: (0, 0)),
            pl.BlockSpec((1, NUM_CODES), lambda t: (0, 0)),
            pl.BlockSpec((TOK_BLK, 1), lambda t: (t, 0)),
            pl.BlockSpec((1, NUM_CODES), lambda t: (0, 0)),
        ],
        out_specs=[
            pl.BlockSpec((TOK_BLK, 1), lambda t: (t, 0)),
            pl.BlockSpec((1, 1), lambda t: (0, 0)),
        ],
        out_shape=[
            jax.ShapeDtypeStruct((N_TOK, 1), jnp.int32),
            jax.ShapeDtypeStruct((1, 1), jnp.float32),
        ],
        scratch_shapes=[pltpu.VMEM((TOK_BLK, NUM_CODES), jnp.float32)],
    )(z_flat, es, e2, z2, colsf)

    idx_flat = min_idx_col.reshape(N_TOK)

    sc_gather = pl.kernel(
        _sc_gather_body,
        out_type=jax.ShapeDtypeStruct((N_TOK, DIM), jnp.float32),
        mesh=plsc.VectorSubcoreMesh(core_axis_name="c", subcore_axis_name="s"),
        scratch_types=[
            pltpu.VMEM((_CHUNK,), jnp.int32),
            pltpu.VMEM((_CHUNK, DIM), jnp.float32),
            pltpu.SemaphoreType.DMA,
        ],
    )
    z_q = sc_gather(e, idx_flat).reshape(z.shape)

    valid_count = jnp.sum(mask) * z.shape[-1]
    commitment_loss = dsum[0, 0] / valid_count
    min_idx = idx_flat.reshape(z.shape[0], z.shape[1])
    return commitment_loss, z_q, min_idx


# speedup vs baseline: 1.4203x; 1.0583x over previous
import jax
import jax.numpy as jnp
from jax import lax
from jax.experimental import pallas as pl
from jax.experimental.pallas import tpu as pltpu
from jax.experimental.pallas import tpu_sc as plsc

NUM_CODES = 8192
DIM = 256
N_TOK = 16384
TOK_BLK = 512
CODE_BLK = 1024

_NC = 2
_NS = 16
_NW = _NC * _NS
_ROWS_PER_W = N_TOK // _NW
_CHUNK = 128
_NCHUNK = _ROWS_PER_W // _CHUNK


WIN = 2736
N_WIN = 3


def _argmin_body(z_ref, es_ref, e2_ref, z2_ref, cf_ref, idx_out, dsum_out,
                 d_buf, tgt_buf):
    t = pl.program_id(0)
    nt = pl.num_programs(0) - 1
    INF = jnp.float32(jnp.inf)
    slot = lax.rem(t, 2)
    prev = lax.rem(t + 1, 2)

    @pl.when(t == 0)
    def _():
        dsum_out[...] = jnp.zeros((1, 1), jnp.float32)

    @pl.when(t > 0)
    def _():
        run_idx_f = jnp.full((TOK_BLK, 1), INF)
        for c in range(NUM_CODES // CODE_BLK):
            lo, hi = c * CODE_BLK, (c + 1) * CODE_BLK
            d = d_buf[prev, :, lo:hi]
            colsf = cf_ref[:, lo:hi]
            w_first, w_last = lo // WIN, (hi - 1) // WIN
            if w_first == w_last:
                tgt = tgt_buf[prev, w_first]
            else:
                split = jnp.float32(w_last * WIN)
                tgt = jnp.where(colsf < split,
                                tgt_buf[prev, w_first], tgt_buf[prev, w_last])
            cand = jnp.where(d == tgt, colsf, INF)
            run_idx_f = jnp.minimum(run_idx_f,
                                    jnp.min(cand, axis=1, keepdims=True))
        idx_out[...] = run_idx_f.astype(jnp.int32)

    @pl.when(t < nt)
    def _():
        z = z_ref[...]
        z2 = z2_ref[...]
        wmin = [jnp.full((TOK_BLK, 1), INF) for _ in range(N_WIN)]
        for c in range(NUM_CODES // CODE_BLK):
            lo, hi = c * CODE_BLK, (c + 1) * CODE_BLK
            m2 = lax.dot_general(z, es_ref[lo:hi, :], (((1,), (1,)), ((), ())),
                                 preferred_element_type=jnp.float32)
            d = (z2 + e2_ref[:, lo:hi]) + m2
            d_buf[slot, :, lo:hi] = d
            w_first, w_last = lo // WIN, (hi - 1) // WIN
            if w_first == w_last:
                loc = jnp.min(d, axis=1, keepdims=True)
                wmin[w_first] = jnp.minimum(wmin[w_first], loc)
            else:
                split = w_last * WIN
                cols = lax.broadcasted_iota(jnp.int32, (TOK_BLK, CODE_BLK), 1) + lo
                in_lo = cols < split
                loc_a = jnp.min(jnp.where(in_lo, d, INF), axis=1, keepdims=True)
                loc_b = jnp.min(jnp.where(in_lo, INF, d), axis=1, keepdims=True)
                wmin[w_first] = jnp.minimum(wmin[w_first], loc_a)
                wmin[w_last] = jnp.minimum(wmin[w_last], loc_b)

        win = jnp.zeros((TOK_BLK, 1), jnp.int32)
        acc = wmin[0]
        for w in range(1, N_WIN):
            accq = acc.astype(jnp.bfloat16).astype(jnp.float32)
            take = wmin[w] < accq
            win = jnp.where(take, jnp.int32(w), win)
            acc = jnp.where(take, wmin[w], accq)

        for w in range(N_WIN):
            tgt_buf[slot, w] = jnp.where(win == w, wmin[w], INF)

        dsum_out[...] += jnp.sum(acc, keepdims=True)


def _sc_gather_body(table_hbm, idx_hbm, out_hbm, idx_v, rows_v, sem):
    cid = lax.axis_index("c")
    sid = lax.axis_index("s")
    wid = sid * _NC + cid
    for j in range(_NCHUNK):
        base = wid * _ROWS_PER_W + j * _CHUNK
        pltpu.sync_copy(idx_hbm.at[pl.ds(base, _CHUNK)], idx_v)
        pltpu.async_copy(table_hbm.at[idx_v], rows_v, sem).wait()
        pltpu.sync_copy(rows_v, out_hbm.at[pl.ds(base, _CHUNK)])


def kernel(z, codebook, mask):
    e = lax.stop_gradient(codebook)
    z_flat = z.reshape(-1, DIM)
    es = e * jnp.float32(-2.0)
    e2 = jnp.sum(e ** 2, axis=1).reshape(1, NUM_CODES)
    z2 = jnp.sum(z_flat ** 2, axis=1, keepdims=True)
    colsf = lax.iota(jnp.float32, NUM_CODES).reshape(1, NUM_CODES)

    nt = N_TOK // TOK_BLK
    min_idx_col, dsum = pl.pallas_call(
        _argmin_body,
        grid=(nt + 1,),
        in_specs=[
            pl.BlockSpec((TOK_BLK, DIM), lambda t: (lax.min(t, nt - 1), 0)),
            pl.BlockSpec((NUM_CODES, DIM), lambda t: (0, 0)),
            pl.BlockSpec((1, NUM_CODES), lambda t: (0, 0)),
            pl.BlockSpec((TOK_BLK, 1), lambda t: (lax.min(t, nt - 1), 0)),
            pl.BlockSpec((1, NUM_CODES), lambda t: (0, 0)),
        ],
        out_specs=[
            pl.BlockSpec((TOK_BLK, 1), lambda t: (lax.max(t - 1, 0), 0)),
            pl.BlockSpec((1, 1), lambda t: (0, 0)),
        ],
        out_shape=[
            jax.ShapeDtypeStruct((N_TOK, 1), jnp.int32),
            jax.ShapeDtypeStruct((1, 1), jnp.float32),
        ],
        scratch_shapes=[
            pltpu.VMEM((2, TOK_BLK, NUM_CODES), jnp.float32),
            pltpu.VMEM((2, N_WIN, TOK_BLK, 1), jnp.float32),
        ],
    )(z_flat, es, e2, z2, colsf)

    idx_flat = min_idx_col.reshape(N_TOK)

    sc_gather = pl.kernel(
        _sc_gather_body,
        out_type=jax.ShapeDtypeStruct((N_TOK, DIM), jnp.float32),
        mesh=plsc.VectorSubcoreMesh(core_axis_name="c", subcore_axis_name="s"),
        scratch_types=[
            pltpu.VMEM((_CHUNK,), jnp.int32),
            pltpu.VMEM((_CHUNK, DIM), jnp.float32),
            pltpu.SemaphoreType.DMA,
        ],
    )
    z_q = sc_gather(e, idx_flat).reshape(z.shape)

    valid_count = jnp.sum(mask) * z.shape[-1]
    commitment_loss = dsum[0, 0] / valid_count
    min_idx = idx_flat.reshape(z.shape[0], z.shape[1])
    return commitment_loss, z_q, min_idx
